# baseline (speedup 1.0000x reference)
import jax
import jax.numpy as jnp
from jax import lax
from jax.experimental import pallas as pl
from jax.experimental.pallas import tpu as pltpu

CODEBOOK = 8192
DIM = 768
COMMIT = 0.25

NB = 8
BKT = CODEBOOK // NB
NBUF = 8


def _vq_body(z_ref, zr_ref, w_hbm, qst_ref, idx_ref, loss_ref,
             bufs, sems, trow):
    zb = z_ref[...]

    def start(c):
        slot = c % NBUF
        pltpu.make_async_copy(
            w_hbm.at[pl.ds(c * BKT, BKT), :], bufs.at[slot], sems.at[slot]
        ).start()

    for c in range(min(NBUF, NB)):
        start(c)
    best_m = jnp.float32(-jnp.inf)
    best_i = jnp.int32(0)
    for c in range(NB):
        slot = c % NBUF
        pltpu.make_async_copy(
            w_hbm.at[pl.ds(c * BKT, BKT), :], bufs.at[slot], sems.at[slot]
        ).wait()
        if c + NBUF < NB:
            start(c + NBUF)
        wb = bufs[slot]
        dots = lax.dot_general(wb, zb, (((1,), (0,)), ((), ())),
                               preferred_element_type=jnp.float32)
        m = jnp.max(dots)
        iota = lax.broadcasted_iota(jnp.int32, (BKT, 1), 0)
        cand = jnp.where(dots == m, iota, jnp.int32(BKT))
        a = jnp.min(cand)
        better = m > best_m

        @pl.when(better)
        def _():
            trow[...] = bufs[slot, pl.ds(a, 1), :]
        best_i = jnp.where(better, a + c * BKT, best_i)
        best_m = jnp.where(better, m, best_m)
    zrow = zr_ref[0, :]
    d = zrow - trow[0, :]
    qst_ref[0, :] = zrow - d
    loss_ref[0] = jnp.float32(COMMIT) * (jnp.sum(d * d) / jnp.float32(DIM))
    idx_ref[0] = best_i


_vq_call = pl.pallas_call(
    _vq_body,
    in_specs=[
        pl.BlockSpec(memory_space=pltpu.VMEM),
        pl.BlockSpec(memory_space=pltpu.VMEM),
        pl.BlockSpec(memory_space=pl.ANY),
    ],
    out_specs=[
        pl.BlockSpec(memory_space=pltpu.VMEM),
        pl.BlockSpec(memory_space=pltpu.SMEM),
        pl.BlockSpec(memory_space=pltpu.SMEM),
    ],
    out_shape=[
        jax.ShapeDtypeStruct((1, DIM), jnp.float32),
        jax.ShapeDtypeStruct((1,), jnp.int32),
        jax.ShapeDtypeStruct((1,), jnp.float32),
    ],
    scratch_shapes=[
        pltpu.VMEM((NBUF, BKT, DIM), jnp.float32),
        pltpu.SemaphoreType.DMA((NBUF,)),
        pltpu.VMEM((1, DIM), jnp.float32),
    ],
)


def kernel(z, W):
    qst, idxv, lossv = _vq_call(z[:, None], z[None, :], W)
    return qst[0], idxv[0], lossv[0]


# speedup vs baseline: 1.0217x; 1.0217x over previous
import jax
import jax.numpy as jnp
from jax import lax
from jax.experimental import pallas as pl
from jax.experimental.pallas import tpu as pltpu

CODEBOOK = 8192
DIM = 768
COMMIT = 0.25

CHUNKS = ((0, 2048), (2048, 2048), (4096, 2048), (6144, 1024),
          (7168, 1024))


def _vq_body(z_ref, zr_ref, w_hbm, qst_ref, idx_ref, loss_ref,
             b0, b1, b2, b3, b4, sems, trow):
    zb = z_ref[...]
    bufs = (b0, b1, b2, b3, b4)

    for c, (off, n) in enumerate(CHUNKS):
        pltpu.make_async_copy(
            w_hbm.at[pl.ds(off, n), :], bufs[c], sems.at[c]
        ).start()
    best_m = jnp.float32(-jnp.inf)
    best_i = jnp.int32(0)
    for c, (off, n) in enumerate(CHUNKS):
        pltpu.make_async_copy(
            w_hbm.at[pl.ds(off, n), :], bufs[c], sems.at[c]
        ).wait()
        wb = bufs[c][...]
        dots = lax.dot_general(wb, zb, (((1,), (0,)), ((), ())),
                               preferred_element_type=jnp.float32)
        m = jnp.max(dots)
        iota = lax.broadcasted_iota(jnp.int32, (n, 1), 0)
        cand = jnp.where(dots == m, iota, jnp.int32(n))
        a = jnp.min(cand)
        better = m > best_m

        @pl.when(better)
        def _():
            trow[...] = bufs[c][pl.ds(a, 1), :]
        best_i = jnp.where(better, a + off, best_i)
        best_m = jnp.where(better, m, best_m)
    zrow = zr_ref[0, :]
    d = zrow - trow[0, :]
    qst_ref[0, :] = zrow - d
    loss_ref[0] = jnp.float32(COMMIT) * (jnp.sum(d * d) / jnp.float32(DIM))
    idx_ref[0] = best_i


_vq_call = pl.pallas_call(
    _vq_body,
    in_specs=[
        pl.BlockSpec(memory_space=pltpu.VMEM),
        pl.BlockSpec(memory_space=pltpu.VMEM),
        pl.BlockSpec(memory_space=pl.ANY),
    ],
    out_specs=[
        pl.BlockSpec(memory_space=pltpu.VMEM),
        pl.BlockSpec(memory_space=pltpu.SMEM),
        pl.BlockSpec(memory_space=pltpu.SMEM),
    ],
    out_shape=[
        jax.ShapeDtypeStruct((1, DIM), jnp.float32),
        jax.ShapeDtypeStruct((1,), jnp.int32),
        jax.ShapeDtypeStruct((1,), jnp.float32),
    ],
    scratch_shapes=[
        pltpu.VMEM((2048, DIM), jnp.float32),
        pltpu.VMEM((2048, DIM), jnp.float32),
        pltpu.VMEM((2048, DIM), jnp.float32),
        pltpu.VMEM((1024, DIM), jnp.float32),
        pltpu.VMEM((1024, DIM), jnp.float32),
        pltpu.SemaphoreType.DMA((5,)),
        pltpu.VMEM((1, DIM), jnp.float32),
    ],
)


def kernel(z, W):
    qst, idxv, lossv = _vq_call(z[:, None], z[None, :], W)
    return qst[0], idxv[0], lossv[0]


# speedup vs baseline: 1.0496x; 1.0273x over previous
import jax
import jax.numpy as jnp
from jax import lax
from jax.experimental import pallas as pl
from jax.experimental.pallas import tpu as pltpu

CODEBOOK = 8192
DIM = 768
COMMIT = 0.25

NB = 4
BKT = CODEBOOK // NB
NBUF = 4


def _vq_body(z_ref, zr_ref, w_hbm, qst_ref, idx_ref, loss_ref,
             bufs, sems, trow):
    zb = z_ref[...]

    def start(c):
        slot = c % NBUF
        pltpu.make_async_copy(
            w_hbm.at[pl.ds(c * BKT, BKT), :], bufs.at[slot], sems.at[slot]
        ).start()

    for c in range(min(NBUF, NB)):
        start(c)
    best_m = jnp.float32(-jnp.inf)
    best_i = jnp.int32(0)
    for c in range(NB):
        slot = c % NBUF
        pltpu.make_async_copy(
            w_hbm.at[pl.ds(c * BKT, BKT), :], bufs.at[slot], sems.at[slot]
        ).wait()
        if c + NBUF < NB:
            start(c + NBUF)
        wb = bufs[slot]
        dots = lax.dot_general(wb, zb, (((1,), (0,)), ((), ())),
                               preferred_element_type=jnp.float32)
        m = jnp.max(dots)
        iota = lax.broadcasted_iota(jnp.int32, (BKT, 1), 0)
        cand = jnp.where(dots == m, iota, jnp.int32(BKT))
        a = jnp.min(cand)
        better = m > best_m

        @pl.when(better)
        def _():
            trow[...] = bufs[slot, pl.ds(a, 1), :]
        best_i = jnp.where(better, a + c * BKT, best_i)
        best_m = jnp.where(better, m, best_m)
    zrow = zr_ref[0, :]
    d = zrow - trow[0, :]
    qst_ref[0, :] = zrow - d
    loss_ref[0] = jnp.float32(COMMIT) * (jnp.sum(d * d) / jnp.float32(DIM))
    idx_ref[0] = best_i


_vq_call = pl.pallas_call(
    _vq_body,
    in_specs=[
        pl.BlockSpec(memory_space=pltpu.VMEM),
        pl.BlockSpec(memory_space=pltpu.VMEM),
        pl.BlockSpec(memory_space=pl.ANY),
    ],
    out_specs=[
        pl.BlockSpec(memory_space=pltpu.VMEM),
        pl.BlockSpec(memory_space=pltpu.SMEM),
        pl.BlockSpec(memory_space=pltpu.SMEM),
    ],
    out_shape=[
        jax.ShapeDtypeStruct((1, DIM), jnp.float32),
        jax.ShapeDtypeStruct((1,), jnp.int32),
        jax.ShapeDtypeStruct((1,), jnp.float32),
    ],
    scratch_shapes=[
        pltpu.VMEM((NBUF, BKT, DIM), jnp.float32),
        pltpu.SemaphoreType.DMA((NBUF,)),
        pltpu.VMEM((1, DIM), jnp.float32),
    ],
)


def kernel(z, W):
    qst, idxv, lossv = _vq_call(z[:, None], z[None, :], W)
    return qst[0], idxv[0], lossv[0]
